# baseline (speedup 1.0000x reference)
import jax
import jax.numpy as jnp
from jax import lax
from jax.experimental import pallas as pl
from jax.experimental.pallas import tpu as pltpu
from jax.experimental.pallas import tpu_sc as plsc

N_NODES = 10000
D_IN_ = 256
H_ = 4
D_OUT_ = 256
F_ = H_ * D_OUT_
E_ = 160000

NW_ = 32
EPT_ = E_ // NW_
CHUNK_ = 128
EPAD_ = 5120
NCK_ = EPAD_ // CHUNK_
ROWS_ = 10240
TRASH_ = 10000
STRIPE_ = ROWS_ // 16
MEGA_ = 9 * N_NODES


def _matmul_kernel(x_ref, w_ref, o_ref):
    o_ref[...] = jnp.dot(x_ref[...], w_ref[...],
                         preferred_element_type=jnp.float32)


def _proj(x, w):
    n, k = x.shape
    m = w.shape[1]
    blk = 400
    return pl.pallas_call(
        _matmul_kernel,
        grid=(n // blk,),
        in_specs=[pl.BlockSpec((blk, k), lambda i: (i, 0)),
                  pl.BlockSpec((k, m), lambda i: (0, 0))],
        out_specs=pl.BlockSpec((blk, m), lambda i: (i, 0)),
        out_shape=jax.ShapeDtypeStruct((n, m), jnp.float32),
    )(x, w)


def _sc_body(src_hbm, dst_hbm, er_hbm, mega_hbm,
             out_hbm, out2_hbm, w_hbm,
             ids_s, ids_d, wbuf, cbufA, cbufB, sem, acc):
    cid = lax.axis_index("c")
    sid = lax.axis_index("s")
    wid = cid * 16 + sid
    r0 = sid * STRIPE_
    zeros16 = jnp.zeros((16,), jnp.float32)

    def zero_cbufB():
        def zrow(i, _):
            for j in range(8):
                cbufB[i, pl.ds(j * 16, 16)] = zeros16
            return 0
        lax.fori_loop(0, CHUNK_, zrow, 0)

    def wipe_acc():
        for k in range(STRIPE_ // CHUNK_):
            pltpu.sync_copy(cbufB, acc.at[pl.ds(r0 + k * CHUNK_, CHUNK_)])

    for rel in range(2):
        pltpu.sync_copy(src_hbm.at[rel, wid], ids_s)
        pltpu.sync_copy(dst_hbm.at[rel, wid], ids_d)

        zero_cbufB()
        wipe_acc()
        plsc.subcore_barrier()

        def passa(ck, _):
            cbase = ck * CHUNK_
            pltpu.async_copy(mega_hbm.at[ids_s.at[pl.ds(cbase, CHUNK_)]],
                             cbufA, sem).wait()
            pltpu.async_copy(er_hbm.at[ids_d.at[ck]],
                             cbufB, sem).wait()

            def wrow(r, _):
                e = cbufA[r, pl.ds(0, 16)] + cbufB[r, pl.ds(0, 16)]
                e = jnp.maximum(e, 0.2 * e)
                w = jnp.exp(e)
                cbufA[r, pl.ds(0, 16)] = w
                wbuf[pl.ds(r * 16, 16)] = w
                return 0
            lax.fori_loop(0, CHUNK_, wrow, 0)
            pltpu.sync_copy(
                wbuf, w_hbm.at[wid, pl.ds(cbase * 16, CHUNK_ * 16)])
            pltpu.sync_copy(cbufA, acc.at[ids_d.at[ck]], add=True)
            return 0
        lax.fori_loop(0, NCK_, passa, 0)
        plsc.subcore_barrier()
        pltpu.sync_copy(acc.at[pl.ds(r0, STRIPE_)],
                        out2_hbm.at[rel, cid, pl.ds(r0, STRIPE_)])
        plsc.subcore_barrier()

        zero_cbufB()

        for ch in range(2):
            for h in range(H_):
                hidx = jnp.full((16,), 4 * rel + h, jnp.int32)

                def bump(g, _):
                    off = g * 16
                    ids_s[pl.ds(off, 16)] = ids_s[pl.ds(off, 16)] + N_NODES
                    return 0
                lax.fori_loop(0, EPAD_ // 16, bump, 0)
                wipe_acc()
                plsc.subcore_barrier()

                def chunk(ck, _):
                    cbase = ck * CHUNK_
                    pltpu.sync_copy(
                        w_hbm.at[wid, pl.ds(cbase * 16, CHUNK_ * 16)],
                        wbuf)
                    pltpu.async_copy(
                        mega_hbm.at[ids_s.at[pl.ds(cbase, CHUNK_)]],
                        cbufA, sem).wait()

                    def scale(r, _):
                        wgrp = wbuf[pl.ds(r * 16, 16)]
                        wv = lax.gather(
                            wgrp, hidx[:, None],
                            dimension_numbers=lax.GatherDimensionNumbers(
                                offset_dims=(), collapsed_slice_dims=(0,),
                                start_index_map=(0,)),
                            slice_sizes=(1,),
                            mode=lax.GatherScatterMode.PROMISE_IN_BOUNDS)
                        for j in range(8):
                            sl = pl.ds(j * 16, 16)
                            cbufA[r, sl] = cbufA[r, sl] * wv
                        return 0
                    lax.fori_loop(0, CHUNK_, scale, 0)

                    pltpu.sync_copy(cbufA, acc.at[ids_d.at[ck]],
                                    add=True)
                    return 0
                lax.fori_loop(0, NCK_, chunk, 0)
                plsc.subcore_barrier()
                pltpu.sync_copy(
                    acc.at[pl.ds(r0, STRIPE_)],
                    out_hbm.at[rel, cid, h, ch, pl.ds(r0, STRIPE_)])
                plsc.subcore_barrier()


@jax.jit
def _sc_aggregate(src2, dst2, er_tab, mega):
    mesh = plsc.VectorSubcoreMesh(core_axis_name="c", subcore_axis_name="s")
    f = pl.kernel(
        _sc_body,
        out_type=[
            jax.ShapeDtypeStruct((2, 2, H_, 2, ROWS_, 128), jnp.float32),
            jax.ShapeDtypeStruct((2, 2, ROWS_, 128), jnp.float32),
            jax.ShapeDtypeStruct((NW_, EPAD_ * 16), jnp.float32),
        ],
        mesh=mesh,
        scratch_types=[
            pltpu.VMEM((EPAD_,), jnp.int32),
            pltpu.VMEM((NCK_, CHUNK_), jnp.int32),
            pltpu.VMEM((CHUNK_ * 16,), jnp.float32),
            pltpu.VMEM((CHUNK_, 128), jnp.float32),
            pltpu.VMEM((CHUNK_, 128), jnp.float32),
            pltpu.SemaphoreType.DMA,
            pltpu.VMEM_SHARED((ROWS_, 128), jnp.float32),
        ],
    )
    out, out2, _ = f(src2, dst2, er_tab, mega)
    return out, out2


def _mega_block(h_src, W, attn_l, rel):
    Wl = jnp.einsum('khd,hd->kh', W.reshape(D_IN_, H_, D_OUT_), attn_l)
    W_ext = jnp.concatenate(
        [W, jnp.zeros((D_IN_, 4 * rel), jnp.float32), Wl,
         jnp.zeros((D_IN_, 124 - 4 * rel), jnp.float32)], axis=1)
    proj = _proj(h_src, W_ext)
    el_tab = proj[:, F_:]
    feat_tab = (proj[:, :F_].reshape(N_NODES, H_, 2, 128)
                .transpose(2, 1, 0, 3)
                .reshape(2 * H_ * N_NODES, 128))
    return jnp.concatenate([el_tab, feat_tab], axis=0)


def _pad_edges(src, dst, rel):
    src_p = jnp.concatenate(
        [src.astype(jnp.int32).reshape(NW_, EPT_) + rel * MEGA_,
         jnp.full((NW_, EPAD_ - EPT_), rel * MEGA_, jnp.int32)], axis=1)
    dst_p = jnp.concatenate(
        [dst.astype(jnp.int32).reshape(NW_, EPT_),
         jnp.full((NW_, EPAD_ - EPT_), TRASH_, jnp.int32)],
        axis=1).reshape(NW_, NCK_, CHUNK_)
    return src_p, dst_p


def _finish(out_r, out2_r, bias, rel):
    p = (out_r[0] + out_r[1])[:, :, :N_NODES, :]
    p = p.transpose(2, 0, 1, 3).reshape(N_NODES, H_, D_OUT_)
    ssum = (out2_r[0] + out2_r[1])[:N_NODES, 4 * rel:4 * rel + H_]
    rst = p / jnp.maximum(ssum, 1e-9)[:, :, None] + bias[None, :, :]
    return jax.nn.elu(rst).reshape(N_NODES, F_)


def kernel(dst_feat, src_feat_author, src_feat_subject,
           edge_src_writes, edge_dst_writes, edge_src_has, edge_dst_has,
           W_writes, attn_l_writes, attn_r_writes, bias_writes,
           W_has, attn_l_has, attn_r_has, bias_has,
           sem_W1, sem_b1, sem_W2):
    Wr_w = jnp.einsum('khd,hd->kh', W_writes.reshape(D_IN_, H_, D_OUT_),
                      attn_r_writes)
    Wr_h = jnp.einsum('khd,hd->kh', W_has.reshape(D_IN_, H_, D_OUT_),
                      attn_r_has)
    Wr_pad = jnp.concatenate(
        [Wr_w, Wr_h, jnp.zeros((D_IN_, 120), jnp.float32)], axis=1)
    er_tab = _proj(dst_feat, Wr_pad)
    er_tab = jnp.concatenate(
        [er_tab, jnp.zeros((ROWS_ - N_NODES, 128), jnp.float32)], axis=0)

    mega = jnp.concatenate(
        [_mega_block(src_feat_author, W_writes, attn_l_writes, 0),
         _mega_block(src_feat_subject, W_has, attn_l_has, 1)], axis=0)

    sw, dw = _pad_edges(edge_src_writes, edge_dst_writes, 0)
    sh, dh = _pad_edges(edge_src_has, edge_dst_has, 1)
    src2 = jnp.stack([sw, sh], axis=0)
    dst2 = jnp.stack([dw, dh], axis=0)

    out, out2 = _sc_aggregate(src2, dst2, er_tab, mega)
    z_writes = _finish(out[0], out2[0], bias_writes, 0)
    z_has = _finish(out[1], out2[1], bias_has, 1)

    z_r = jnp.stack([z_writes, z_has], axis=1)
    hdn = jnp.tanh(z_r @ sem_W1 + sem_b1)
    w = (hdn @ sem_W2).mean(axis=0)
    a_w = jax.nn.softmax(w, axis=0)
    z = (z_r * a_w[None, :, :]).sum(axis=1)
    att_sc = a_w.squeeze(-1)
    return (z, att_sc)


# speedup vs baseline: 1.1322x; 1.1322x over previous
import jax
import jax.numpy as jnp
from jax import lax
from jax.experimental import pallas as pl
from jax.experimental.pallas import tpu as pltpu
from jax.experimental.pallas import tpu_sc as plsc

N_NODES = 10000
D_IN_ = 256
H_ = 4
D_OUT_ = 256
F_ = H_ * D_OUT_
E_ = 160000

NW_ = 32
EPT_ = E_ // NW_
CHUNK_ = 64
EPAD_ = 5056
NCK_ = EPAD_ // CHUNK_
ROWS_ = 10240
TRASH_ = 10000
STRIPE_ = ROWS_ // 16
MEGA_ = 9 * N_NODES


def _matmul_kernel(x_ref, w_ref, o_ref):
    o_ref[...] = jnp.dot(x_ref[...], w_ref[...],
                         preferred_element_type=jnp.float32)


def _proj(x, w):
    n, k = x.shape
    m = w.shape[1]
    blk = 400
    return pl.pallas_call(
        _matmul_kernel,
        grid=(n // blk,),
        in_specs=[pl.BlockSpec((blk, k), lambda i: (i, 0)),
                  pl.BlockSpec((k, m), lambda i: (0, 0))],
        out_specs=pl.BlockSpec((blk, m), lambda i: (i, 0)),
        out_shape=jax.ShapeDtypeStruct((n, m), jnp.float32),
    )(x, w)


def _sc_body(src_hbm, dst_hbm, er_hbm, mega_hbm,
             out_hbm, out2_hbm, w_hbm,
             ids_s, ids_d, wbuf, cbufA, cbufB, sem, acc):
    cid = lax.axis_index("c")
    sid = lax.axis_index("s")
    wid = cid * 16 + sid
    r0 = sid * STRIPE_
    zeros16 = jnp.zeros((16,), jnp.float32)

    def zero_cbufB():
        def zrow(i, _):
            for j in range(8):
                cbufB[i, pl.ds(j * 16, 16)] = zeros16
            return 0
        lax.fori_loop(0, CHUNK_, zrow, 0)

    def wipe_acc():
        for k in range(STRIPE_ // CHUNK_):
            pltpu.sync_copy(cbufB, acc.at[pl.ds(r0 + k * CHUNK_, CHUNK_)])
        if STRIPE_ % CHUNK_:
            pltpu.sync_copy(
                cbufB.at[pl.ds(0, STRIPE_ % CHUNK_)],
                acc.at[pl.ds(r0 + (STRIPE_ // CHUNK_) * CHUNK_,
                             STRIPE_ % CHUNK_)])

    for rel in range(2):
        pltpu.sync_copy(src_hbm.at[rel, wid], ids_s)
        pltpu.sync_copy(dst_hbm.at[rel, wid], ids_d)

        zero_cbufB()
        wipe_acc()
        plsc.subcore_barrier()

        def passa(ck, _):
            cbase = ck * CHUNK_
            pltpu.async_copy(mega_hbm.at[ids_s.at[pl.ds(cbase, CHUNK_)]],
                             cbufA, sem).wait()
            pltpu.async_copy(er_hbm.at[ids_d.at[ck]],
                             cbufB, sem).wait()

            def wrow(r, _):
                e = cbufA[r, pl.ds(0, 16)] + cbufB[r, pl.ds(0, 16)]
                e = jnp.maximum(e, 0.2 * e)
                w = jnp.exp(e)
                cbufA[r, pl.ds(0, 16)] = w
                wbuf[pl.ds(r * 16, 16)] = w
                return 0
            lax.fori_loop(0, CHUNK_, wrow, 0)
            pltpu.sync_copy(
                wbuf, w_hbm.at[wid, pl.ds(cbase * 16, CHUNK_ * 16)])
            pltpu.sync_copy(cbufA, acc.at[ids_d.at[ck]], add=True)
            return 0
        lax.fori_loop(0, NCK_, passa, 0)
        plsc.subcore_barrier()
        pltpu.sync_copy(acc.at[pl.ds(r0, STRIPE_)],
                        out2_hbm.at[rel, cid, pl.ds(r0, STRIPE_)])
        plsc.subcore_barrier()

        zero_cbufB()

        for ch in range(2):
            for h in range(H_):
                hidx = jnp.full((16,), 4 * rel + h, jnp.int32)

                def bump(g, _):
                    off = g * 16
                    ids_s[pl.ds(off, 16)] = ids_s[pl.ds(off, 16)] + N_NODES
                    return 0
                lax.fori_loop(0, EPAD_ // 16, bump, 0)
                wipe_acc()
                plsc.subcore_barrier()

                def chunk(ck, _):
                    cbase = ck * CHUNK_
                    pltpu.sync_copy(
                        w_hbm.at[wid, pl.ds(cbase * 16, CHUNK_ * 16)],
                        wbuf)
                    pltpu.async_copy(
                        mega_hbm.at[ids_s.at[pl.ds(cbase, CHUNK_)]],
                        cbufA, sem).wait()

                    def scale(r, _):
                        wgrp = wbuf[pl.ds(r * 16, 16)]
                        wv = lax.gather(
                            wgrp, hidx[:, None],
                            dimension_numbers=lax.GatherDimensionNumbers(
                                offset_dims=(), collapsed_slice_dims=(0,),
                                start_index_map=(0,)),
                            slice_sizes=(1,),
                            mode=lax.GatherScatterMode.PROMISE_IN_BOUNDS)
                        for j in range(8):
                            sl = pl.ds(j * 16, 16)
                            cbufA[r, sl] = cbufA[r, sl] * wv
                        return 0
                    lax.fori_loop(0, CHUNK_, scale, 0)

                    pltpu.sync_copy(cbufA, acc.at[ids_d.at[ck]],
                                    add=True)
                    return 0
                lax.fori_loop(0, NCK_, chunk, 0)
                plsc.subcore_barrier()
                pltpu.sync_copy(
                    acc.at[pl.ds(r0, STRIPE_)],
                    out_hbm.at[rel, cid, h, ch, pl.ds(r0, STRIPE_)])
                plsc.subcore_barrier()


@jax.jit
def _sc_aggregate(src2, dst2, er_tab, mega):
    mesh = plsc.VectorSubcoreMesh(core_axis_name="c", subcore_axis_name="s")
    f = pl.kernel(
        _sc_body,
        out_type=[
            jax.ShapeDtypeStruct((2, 2, H_, 2, ROWS_, 128), jnp.float32),
            jax.ShapeDtypeStruct((2, 2, ROWS_, 128), jnp.float32),
            jax.ShapeDtypeStruct((NW_, EPAD_ * 16), jnp.float32),
        ],
        mesh=mesh,
        scratch_types=[
            pltpu.VMEM((EPAD_,), jnp.int32),
            pltpu.VMEM((NCK_, CHUNK_), jnp.int32),
            pltpu.VMEM((CHUNK_ * 16,), jnp.float32),
            pltpu.VMEM((CHUNK_, 128), jnp.float32),
            pltpu.VMEM((CHUNK_, 128), jnp.float32),
            pltpu.SemaphoreType.DMA,
            pltpu.VMEM_SHARED((ROWS_, 128), jnp.float32),
        ],
    )
    out, out2, _ = f(src2, dst2, er_tab, mega)
    return out, out2


def _mega_block(h_src, W, attn_l, rel):
    Wl = jnp.einsum('khd,hd->kh', W.reshape(D_IN_, H_, D_OUT_), attn_l)
    W_ext = jnp.concatenate(
        [W, jnp.zeros((D_IN_, 4 * rel), jnp.float32), Wl,
         jnp.zeros((D_IN_, 124 - 4 * rel), jnp.float32)], axis=1)
    proj = _proj(h_src, W_ext)
    el_tab = proj[:, F_:]
    feat_tab = (proj[:, :F_].reshape(N_NODES, H_, 2, 128)
                .transpose(2, 1, 0, 3)
                .reshape(2 * H_ * N_NODES, 128))
    return jnp.concatenate([el_tab, feat_tab], axis=0)


def _pad_edges(src, dst, rel):
    src_p = jnp.concatenate(
        [src.astype(jnp.int32).reshape(NW_, EPT_) + rel * MEGA_,
         jnp.full((NW_, EPAD_ - EPT_), rel * MEGA_, jnp.int32)], axis=1)
    dst_p = jnp.concatenate(
        [dst.astype(jnp.int32).reshape(NW_, EPT_),
         jnp.full((NW_, EPAD_ - EPT_), TRASH_, jnp.int32)],
        axis=1).reshape(NW_, NCK_, CHUNK_)
    return src_p, dst_p


def _finish(out_r, out2_r, bias, rel):
    p = (out_r[0] + out_r[1])[:, :, :N_NODES, :]
    p = p.transpose(2, 0, 1, 3).reshape(N_NODES, H_, D_OUT_)
    ssum = (out2_r[0] + out2_r[1])[:N_NODES, 4 * rel:4 * rel + H_]
    rst = p / jnp.maximum(ssum, 1e-9)[:, :, None] + bias[None, :, :]
    return jax.nn.elu(rst).reshape(N_NODES, F_)


def kernel(dst_feat, src_feat_author, src_feat_subject,
           edge_src_writes, edge_dst_writes, edge_src_has, edge_dst_has,
           W_writes, attn_l_writes, attn_r_writes, bias_writes,
           W_has, attn_l_has, attn_r_has, bias_has,
           sem_W1, sem_b1, sem_W2):
    Wr_w = jnp.einsum('khd,hd->kh', W_writes.reshape(D_IN_, H_, D_OUT_),
                      attn_r_writes)
    Wr_h = jnp.einsum('khd,hd->kh', W_has.reshape(D_IN_, H_, D_OUT_),
                      attn_r_has)
    Wr_pad = jnp.concatenate(
        [Wr_w, Wr_h, jnp.zeros((D_IN_, 120), jnp.float32)], axis=1)
    er_tab = _proj(dst_feat, Wr_pad)
    er_tab = jnp.concatenate(
        [er_tab, jnp.zeros((ROWS_ - N_NODES, 128), jnp.float32)], axis=0)

    mega = jnp.concatenate(
        [_mega_block(src_feat_author, W_writes, attn_l_writes, 0),
         _mega_block(src_feat_subject, W_has, attn_l_has, 1)], axis=0)

    sw, dw = _pad_edges(edge_src_writes, edge_dst_writes, 0)
    sh, dh = _pad_edges(edge_src_has, edge_dst_has, 1)
    src2 = jnp.stack([sw, sh], axis=0)
    dst2 = jnp.stack([dw, dh], axis=0)

    out, out2 = _sc_aggregate(src2, dst2, er_tab, mega)
    z_writes = _finish(out[0], out2[0], bias_writes, 0)
    z_has = _finish(out[1], out2[1], bias_has, 1)

    z_r = jnp.stack([z_writes, z_has], axis=1)
    hdn = jnp.tanh(z_r @ sem_W1 + sem_b1)
    w = (hdn @ sem_W2).mean(axis=0)
    a_w = jax.nn.softmax(w, axis=0)
    z = (z_r * a_w[None, :, :]).sum(axis=1)
    att_sc = a_w.squeeze(-1)
    return (z, att_sc)
